# Initial kernel scaffold; baseline (speedup 1.0000x reference)
import jax
import jax.numpy as jnp
from jax.experimental import pallas as pl


def kernel(sc_features, fc_features, adj_sc, adj_fc, labels, dropout, W_fc, Wx, Wh, b_lstm, W_sc):
    raise NotImplementedError("write your pallas kernel here")


# speedup vs baseline: 1.8310x; 1.8310x over previous
import jax
import jax.numpy as jnp
from jax import lax
from jax.experimental import pallas as pl
from jax.experimental.pallas import tpu as pltpu

_B, _T, _N, _F, _H2, _H3, _H1 = 32, 20, 90, 90, 16, 16, 32
_U = _N * _H3
_D = _N * _H2
_G = 4 * _U


def _gcn_kernel(fc_ref, adj_ref, w_ref, out_ref):
    w = w_ref[...]
    for t in range(_T):
        xw = jnp.dot(fc_ref[t], w, preferred_element_type=jnp.float32)
        h2 = jnp.maximum(
            jnp.dot(adj_ref[t], xw, preferred_element_type=jnp.float32), 0.0)
        out_ref[t, 0] = h2


def _xproj_kernel(x_ref, wx_ref, b_ref, out_ref):
    out_ref[...] = jnp.dot(
        x_ref[...], wx_ref[...], preferred_element_type=jnp.float32) + b_ref[...]


def _lstm_kernel(xp_ref, wh_ref, out_ref, h_s, c_s):
    t = pl.program_id(0)

    @pl.when(t == 0)
    def _init():
        h_s[...] = jnp.zeros_like(h_s)
        c_s[...] = jnp.zeros_like(c_s)

    z = xp_ref[0] + jnp.dot(
        h_s[...].astype(wh_ref.dtype), wh_ref[...],
        preferred_element_type=jnp.float32)
    i = jax.nn.sigmoid(z[:, :_U])
    f = jax.nn.sigmoid(z[:, _U:2 * _U])
    g = jnp.tanh(z[:, 2 * _U:3 * _U])
    o = jax.nn.sigmoid(z[:, 3 * _U:])
    c = f * c_s[...] + i * g
    h = o * jnp.tanh(c)
    c_s[...] = c
    h_s[...] = h

    @pl.when(t == _T - 1)
    def _emit():
        out_ref[...] = h


def _dec_kernel(h_ref, adj_ref, w_ref, out_ref):
    y = jnp.dot(h_ref[0], w_ref[...], preferred_element_type=jnp.float32)
    h1 = jnp.maximum(
        jnp.dot(adj_ref[...], y, preferred_element_type=jnp.float32), 0.0)
    r = jnp.maximum(
        lax.dot_general(h1, h1, (((1,), (1,)), ((), ())),
                        preferred_element_type=jnp.float32), 0.0)
    ri = lax.broadcasted_iota(jnp.int32, (_N, _N), 0)
    ci = lax.broadcasted_iota(jnp.int32, (_N, _N), 1)
    out_ref[0] = jnp.where(ri == ci, 1.0, r)


def kernel(sc_features, fc_features, adj_sc, adj_fc, labels, dropout,
           W_fc, Wx, Wh, b_lstm, W_sc):
    h2p = pl.pallas_call(
        _gcn_kernel,
        grid=(_B,),
        in_specs=[
            pl.BlockSpec((_T, _N, _F), lambda b: (b, 0, 0)),
            pl.BlockSpec((_T, _N, _N), lambda b: (b, 0, 0)),
            pl.BlockSpec((_F, _H2), lambda b: (0, 0)),
        ],
        out_specs=pl.BlockSpec((_T, 1, _N, _H2), lambda b: (0, b, 0, 0)),
        out_shape=jax.ShapeDtypeStruct((_T, _B, _N, _H2), jnp.float32),
    )(fc_features, adj_fc, W_fc)

    x = h2p.reshape(_T * _B, _D)
    xproj = pl.pallas_call(
        _xproj_kernel,
        grid=(5, 5),
        in_specs=[
            pl.BlockSpec((128, _D), lambda j, i: (i, 0)),
            pl.BlockSpec((_D, 1152), lambda j, i: (0, j)),
            pl.BlockSpec((1, 1152), lambda j, i: (0, j)),
        ],
        out_specs=pl.BlockSpec((128, 1152), lambda j, i: (i, j)),
        out_shape=jax.ShapeDtypeStruct((_T * _B, _G), jnp.float32),
    )(x, Wx, b_lstm.reshape(1, _G))

    xp = xproj.reshape(_T, _B, _G)
    h = pl.pallas_call(
        _lstm_kernel,
        grid=(_T,),
        in_specs=[
            pl.BlockSpec((1, _B, _G), lambda t: (t, 0, 0)),
            pl.BlockSpec((_U, _G), lambda t: (0, 0)),
        ],
        out_specs=pl.BlockSpec((_B, _U), lambda t: (0, 0)),
        out_shape=jax.ShapeDtypeStruct((_B, _U), jnp.float32),
        scratch_shapes=[pltpu.VMEM((_B, _U), jnp.float32),
                        pltpu.VMEM((_B, _U), jnp.float32)],
    )(xp, Wh)

    lstm_h = h.reshape(_B, _N, _H3)
    rec = pl.pallas_call(
        _dec_kernel,
        grid=(_B,),
        in_specs=[
            pl.BlockSpec((1, _N, _H3), lambda b: (b, 0, 0)),
            pl.BlockSpec((_N, _N), lambda b: (0, 0)),
            pl.BlockSpec((_H3, _H1), lambda b: (0, 0)),
        ],
        out_specs=pl.BlockSpec((1, _N, _N), lambda b: (b, 0, 0)),
        out_shape=jax.ShapeDtypeStruct((_B, _N, _N), jnp.float32),
    )(lstm_h, adj_sc, W_sc)
    return rec.reshape(_B, _N * _N)
